# baseline (speedup 1.0000x reference)
import functools

import jax
import jax.numpy as jnp
from jax import lax
from jax.experimental import pallas as pl
from jax.experimental.pallas import tpu as pltpu
from jax.experimental.pallas import tpu_sc as plsc

_NC = 2
_NS = 16
_NW = _NC * _NS
_L = 16

_R = 128
_K = 64
_M = 512
_NK = _M // _K
_RP = 16544


def _scalar(vec16):
    return jnp.max(vec16)


def _sc_body(bpw, jch, batch,
             xt_hbm, meta_hbm, rowptr_hbm, out_hbm,
             rowptr_v, meta_v, xr0, xr1, acc_v, g0, g1):
    cid = lax.axis_index("c")
    sid = lax.axis_index("s")
    wid = sid * _NC + cid

    zeros16 = jnp.zeros((_L,), jnp.float32)
    xr = (xr0, xr1)
    gsem = (g0, g1)

    def issue_gather(k, buf):
        pltpu.async_copy(
            xt_hbm.at[meta_v.at[0, pl.ds(k * _K, _K)]], xr[buf], gsem[buf])

    def wait_gather(k, buf):
        pltpu.make_async_copy(
            xt_hbm.at[meta_v.at[0, pl.ds(k * _K, _K)]], xr[buf],
            gsem[buf]).wait()

    pltpu.sync_copy(rowptr_hbm, rowptr_v)

    def block_body(bi, carry):
        blk = wid * bpw + bi
        base_row = blk * _R

        def zbody(i, zc):
            acc_v[pl.ds(i * _L, _L)] = zeros16
            return zc
        lax.fori_loop(0, _R * batch // _L, zbody, 0)

        e_start = _scalar(plsc.load_gather(
            rowptr_v, [jnp.full((_L,), base_row, jnp.int32)]))
        e_end = _scalar(plsc.load_gather(
            rowptr_v, [jnp.full((_L,), base_row + _R, jnp.int32)]))
        m0 = e_start // _M
        m1 = (e_end + _M - 1) // _M

        def group_body(m, gc):
            mbase = m * _M
            pltpu.sync_copy(meta_hbm.at[:, pl.ds(mbase, _M)], meta_v)
            issue_gather(0, 0)

            for k in range(_NK):
                buf = k & 1
                if k + 1 < _NK:
                    issue_gather(k + 1, 1 - buf)
                wait_gather(k, buf)
                xrb = xr[buf]

                ck0 = mbase + k * _K
                ck1e = jnp.minimum(ck0 + _K, e_end)
                es0 = jnp.maximum(ck0, e_start)

                fi = es0 - mbase
                r_first = _scalar(plsc.load_gather(
                    meta_v, [jnp.full((_L,), 1, jnp.int32),
                             jnp.full((_L,), fi, jnp.int32)])) - base_row

                def row_cond(state):
                    _, es = state
                    return es < ck1e

                def row_body(state):
                    r, es = state
                    ee = jnp.minimum(_scalar(plsc.load_gather(
                        rowptr_v, [jnp.full((_L,), base_row + r + 1, jnp.int32)])), ck1e)

                    def ebody(e, accs):
                        le = e - ck0
                        gi = e - mbase
                        v16 = plsc.bitcast(plsc.load_gather(
                            meta_v, [jnp.full((_L,), 2, jnp.int32),
                                     jnp.full((_L,), gi, jnp.int32)]),
                            jnp.float32)
                        return tuple(
                            accs[j] + xrb[le, pl.ds(j * _L, _L)] * v16
                            for j in range(jch))
                    accs = plsc.parallel_loop(
                        es, ee, 1, unroll=2,
                        carry=tuple(zeros16 for _ in range(jch)))(ebody)

                    rybase = r * batch
                    for j in range(jch):
                        plsc.addupdate(
                            acc_v.at[pl.ds(rybase + j * _L, _L)], accs[j])
                    return (r + 1, ee)

                lax.while_loop(row_cond, row_body, (r_first, es0))
            return gc
        lax.fori_loop(m0, m1, group_body, 0)

        pltpu.sync_copy(acc_v.at[pl.ds(0, _R * batch)],
                        out_hbm.at[pl.ds(base_row * batch, _R * batch)])
        return carry
    lax.fori_loop(0, bpw, block_body, 0)


@functools.partial(jax.jit, static_argnums=(3, 4, 5))
def _sc_spmm(xt, meta_p, rowptr_p, n_items, batch, nblk):
    bpw = nblk // _NW
    jch = batch // _L
    mesh = plsc.VectorSubcoreMesh(core_axis_name="c", subcore_axis_name="s")
    body = functools.partial(_sc_body, bpw, jch, batch)
    return pl.kernel(
        body,
        out_type=jax.ShapeDtypeStruct((n_items * batch,), jnp.float32),
        mesh=mesh,
        compiler_params=pltpu.CompilerParams(needs_layout_passes=False),
        scratch_types=[
            pltpu.VMEM((_RP,), jnp.int32),
            pltpu.VMEM((3, _M), jnp.int32),
            pltpu.VMEM((_K, batch), jnp.float32),
            pltpu.VMEM((_K, batch), jnp.float32),
            pltpu.VMEM((_R * batch,), jnp.float32),
            pltpu.SemaphoreType.DMA,
            pltpu.SemaphoreType.DMA,
        ],
    )(xt, meta_p, rowptr_p)


def kernel(X, S_rows, S_cols, S_vals):
    batch, n_items = X.shape
    nnz = S_rows.shape[0]
    nblk = n_items // _R

    xt = X.T

    npad = -nnz % _M
    cols_p = jnp.pad(S_cols, (0, npad))
    rows_p = jnp.pad(S_rows, (0, npad), constant_values=n_items)
    vals_p = jnp.pad(S_vals, (0, npad))
    meta_p = jnp.stack([cols_p.astype(jnp.int32),
                        rows_p.astype(jnp.int32),
                        lax.bitcast_convert_type(vals_p, jnp.int32)])

    grid = jnp.arange(n_items + 1, dtype=jnp.int32).astype(S_rows.dtype)
    rowptr = jnp.searchsorted(S_rows, grid, side="left").astype(jnp.int32)
    rowptr_p = jnp.pad(rowptr, (0, _RP - (n_items + 1)),
                       constant_values=nnz)

    out_flat = _sc_spmm(xt, meta_p, rowptr_p, n_items, batch, nblk)
    return out_flat.reshape(n_items, batch).T


# speedup vs baseline: 4.2644x; 4.2644x over previous
import functools

import jax
import jax.numpy as jnp
from jax import lax
from jax.experimental import pallas as pl
from jax.experimental.pallas import tpu as pltpu
from jax.experimental.pallas import tpu_sc as plsc

_NC = 2
_NS = 16
_NW = _NC * _NS
_L = 16

_R = 128
_K = 64
_M = 1024
_NK = _M // _K


def _sc_body(bpw, jch, batch,
             xt_hbm, meta_hbm, bounds_hbm, out_hbm,
             bounds_v, meta_v, xr0, xr1, rbase_v, acc_v, g0, g1):
    cid = lax.axis_index("c")
    sid = lax.axis_index("s")
    wid = sid * _NC + cid

    pltpu.sync_copy(bounds_hbm, bounds_v)

    iota16 = lax.iota(jnp.int32, _L)
    zeros16 = jnp.zeros((_L,), jnp.float32)
    xr = (xr0, xr1)
    gsem = (g0, g1)

    def issue_gather(k, buf):
        pltpu.async_copy(
            xt_hbm.at[meta_v.at[0, pl.ds(k * _K, _K)]], xr[buf], gsem[buf])

    def wait_gather(k, buf):
        pltpu.make_async_copy(
            xt_hbm.at[meta_v.at[0, pl.ds(k * _K, _K)]], xr[buf],
            gsem[buf]).wait()

    def block_body(bi, carry):
        blk = wid * bpw + bi
        base_row = blk * _R

        def zbody(i, zc):
            acc_v[pl.ds(i * _L, _L)] = zeros16
            return zc
        lax.fori_loop(0, _R * batch // _L, zbody, 0)

        eb0 = plsc.load_gather(bounds_v, [jnp.full((_L,), blk, jnp.int32)])
        eb1 = plsc.load_gather(bounds_v, [jnp.full((_L,), blk + 1, jnp.int32)])
        e_start = jnp.max(eb0)
        e_end = jnp.max(eb1)
        m0 = e_start // _M
        m1 = (e_end + _M - 1) // _M

        def group_body(m, gc):
            mbase = m * _M
            pltpu.sync_copy(meta_hbm.at[:, pl.ds(mbase, _M)], meta_v)
            issue_gather(0, 0)

            for k in range(_NK):
                buf = k & 1
                if k + 1 < _NK:
                    issue_gather(k + 1, 1 - buf)
                wait_gather(k, buf)
                xrb = xr[buf]

                for g in range(_K // _L):
                    rows16 = meta_v[1, pl.ds(k * _K + g * _L, _L)]
                    ge16 = (mbase + k * _K + g * _L) + iota16
                    ok = jnp.logical_and(ge16 >= e_start, ge16 < e_end)
                    rloc = jnp.clip(rows16 - base_row, 0, _R - 1)
                    rbase_v[pl.ds(g * _L, _L)] = (
                        jnp.where(ok, rloc, _R) * batch)

                half = jch // 2
                @plsc.parallel_loop(0, 2 * _K, 1, unroll=2)
                def scale_body(i):
                    e = i >> 1
                    h = (i & 1) * half
                    li = jnp.full((_L,), k * _K + e, jnp.int32)
                    v16i = plsc.load_gather(
                        meta_v, [jnp.full((_L,), 2, jnp.int32), li])
                    v16 = plsc.bitcast(v16i, jnp.float32)
                    b16 = plsc.load_gather(rbase_v, [jnp.full((_L,), e, jnp.int32)])
                    addr0 = b16 + iota16 + (h * _L)
                    ys = [xrb[e, pl.ds((h + j) * _L, _L)] * v16
                          for j in range(half)]
                    for j in range(half):
                        plsc.addupdate_scatter(
                            acc_v, [addr0 + (j * _L)], ys[j])
            return gc
        lax.fori_loop(m0, m1, group_body, 0)

        pltpu.sync_copy(acc_v.at[pl.ds(0, _R * batch)],
                        out_hbm.at[pl.ds(base_row * batch, _R * batch)])
        return carry
    lax.fori_loop(0, bpw, block_body, 0)


@functools.partial(jax.jit, static_argnums=(3, 4, 5))
def _sc_spmm(xt, meta_p, bounds_p, n_items, batch, nblk):
    bpw = nblk // _NW
    jch = batch // _L
    mesh = plsc.VectorSubcoreMesh(core_axis_name="c", subcore_axis_name="s")
    body = functools.partial(_sc_body, bpw, jch, batch)
    return pl.kernel(
        body,
        out_type=jax.ShapeDtypeStruct((n_items * batch,), jnp.float32),
        mesh=mesh,
        compiler_params=pltpu.CompilerParams(needs_layout_passes=False),
        scratch_types=[
            pltpu.VMEM((bounds_p.shape[0],), jnp.int32),
            pltpu.VMEM((3, _M), jnp.int32),
            pltpu.VMEM((_K, batch), jnp.float32),
            pltpu.VMEM((_K, batch), jnp.float32),
            pltpu.VMEM((_K,), jnp.int32),
            pltpu.VMEM(((_R + 1) * batch,), jnp.float32),
            pltpu.SemaphoreType.DMA,
            pltpu.SemaphoreType.DMA,
        ],
    )(xt, meta_p, bounds_p)


def kernel(X, S_rows, S_cols, S_vals):
    batch, n_items = X.shape
    nnz = S_rows.shape[0]
    nblk = n_items // _R

    xt = X.T

    npad = -nnz % _M
    cols_p = jnp.pad(S_cols, (0, npad))
    rows_p = jnp.pad(S_rows, (0, npad), constant_values=n_items)
    vals_p = jnp.pad(S_vals, (0, npad))
    meta_p = jnp.stack([cols_p.astype(jnp.int32),
                        rows_p.astype(jnp.int32),
                        lax.bitcast_convert_type(vals_p, jnp.int32)])

    grid = (jnp.arange(nblk + 1, dtype=jnp.int32) * _R).astype(S_rows.dtype)
    bounds = jnp.searchsorted(S_rows, grid, side="left").astype(jnp.int32)
    bpad = (-(nblk + 1)) % 16
    bounds_p = jnp.pad(bounds, (0, bpad), constant_values=nnz)

    out_flat = _sc_spmm(xt, meta_p, bounds_p, n_items, batch, nblk)
    return out_flat.reshape(n_items, batch).T


# speedup vs baseline: 5.1082x; 1.1979x over previous
import functools

import jax
import jax.numpy as jnp
from jax import lax
from jax.experimental import pallas as pl
from jax.experimental.pallas import tpu as pltpu
from jax.experimental.pallas import tpu_sc as plsc

_NC = 2
_NS = 16
_NW = _NC * _NS
_L = 16

_R = 128
_K = 64
_M = 512
_NK = _M // _K


def _sc_body(bpw, jch, batch,
             xt_hbm, meta_hbm, bounds_hbm, out_hbm,
             bounds_v, meta_v, xr0, xr1, rbase_v, acc_v, g0, g1):
    cid = lax.axis_index("c")
    sid = lax.axis_index("s")
    wid = sid * _NC + cid

    pltpu.sync_copy(bounds_hbm, bounds_v)

    iota16 = lax.iota(jnp.int32, _L)
    zeros16 = jnp.zeros((_L,), jnp.float32)
    xr = (xr0, xr1)
    gsem = (g0, g1)

    def issue_gather(k, buf):
        pltpu.async_copy(
            xt_hbm.at[meta_v.at[0, pl.ds(k * _K, _K)]], xr[buf], gsem[buf])

    def wait_gather(k, buf):
        pltpu.make_async_copy(
            xt_hbm.at[meta_v.at[0, pl.ds(k * _K, _K)]], xr[buf],
            gsem[buf]).wait()

    def block_body(bi, carry):
        blk = wid * bpw + bi
        base_row = blk * _R

        def zbody(i, zc):
            acc_v[pl.ds(i * _L, _L)] = zeros16
            return zc
        lax.fori_loop(0, _R * batch // _L, zbody, 0)

        eb0 = plsc.load_gather(bounds_v, [jnp.full((_L,), blk, jnp.int32)])
        eb1 = plsc.load_gather(bounds_v, [jnp.full((_L,), blk + 1, jnp.int32)])
        e_start = jnp.max(eb0)
        e_end = jnp.max(eb1)
        m0 = e_start // _M
        m1 = (e_end + _M - 1) // _M

        def group_body(m, gc):
            mbase = m * _M
            pltpu.sync_copy(meta_hbm.at[:, pl.ds(mbase, _M)], meta_v)

            def live(k):
                return jnp.logical_and(mbase + k * _K < e_end,
                                       mbase + (k + 1) * _K > e_start)

            pl.when(live(0))(lambda: issue_gather(0, 0))

            for k in range(_NK):
                buf = k & 1
                if k + 1 < _NK:
                    pl.when(live(k + 1))(
                        functools.partial(issue_gather, k + 1, 1 - buf))

                def do_chunk(k=k, buf=buf):
                    wait_gather(k, buf)
                    xrb = xr[buf]

                    for g in range(_K // _L):
                        rows16 = meta_v[1, pl.ds(k * _K + g * _L, _L)]
                        ge16 = (mbase + k * _K + g * _L) + iota16
                        ok = jnp.logical_and(ge16 >= e_start, ge16 < e_end)
                        rloc = jnp.clip(rows16 - base_row, 0, _R - 1)
                        rbase_v[pl.ds(g * _L, _L)] = (
                            jnp.where(ok, rloc, _R) * batch)

                    half = jch // 2
                    @plsc.parallel_loop(0, 2 * _K, 1, unroll=2)
                    def scale_body(i):
                        e = i >> 1
                        h = (i & 1) * half
                        li = jnp.full((_L,), k * _K + e, jnp.int32)
                        v16i = plsc.load_gather(
                            meta_v, [jnp.full((_L,), 2, jnp.int32), li])
                        v16 = plsc.bitcast(v16i, jnp.float32)
                        b16 = plsc.load_gather(
                            rbase_v, [jnp.full((_L,), e, jnp.int32)])
                        addr0 = b16 + iota16 + (h * _L)
                        ys = [xrb[e, pl.ds((h + j) * _L, _L)] * v16
                              for j in range(half)]
                        for j in range(half):
                            plsc.addupdate_scatter(
                                acc_v, [addr0 + (j * _L)], ys[j])

                pl.when(live(k))(do_chunk)
            return gc
        lax.fori_loop(m0, m1, group_body, 0)

        pltpu.sync_copy(acc_v.at[pl.ds(0, _R * batch)],
                        out_hbm.at[pl.ds(base_row * batch, _R * batch)])
        return carry
    lax.fori_loop(0, bpw, block_body, 0)


@functools.partial(jax.jit, static_argnums=(3, 4, 5))
def _sc_spmm(xt, meta_p, bounds_p, n_items, batch, nblk):
    bpw = nblk // _NW
    jch = batch // _L
    mesh = plsc.VectorSubcoreMesh(core_axis_name="c", subcore_axis_name="s")
    body = functools.partial(_sc_body, bpw, jch, batch)
    return pl.kernel(
        body,
        out_type=jax.ShapeDtypeStruct((n_items * batch,), jnp.float32),
        mesh=mesh,
        compiler_params=pltpu.CompilerParams(needs_layout_passes=False),
        scratch_types=[
            pltpu.VMEM((bounds_p.shape[0],), jnp.int32),
            pltpu.VMEM((3, _M), jnp.int32),
            pltpu.VMEM((_K, batch), jnp.float32),
            pltpu.VMEM((_K, batch), jnp.float32),
            pltpu.VMEM((_K,), jnp.int32),
            pltpu.VMEM(((_R + 1) * batch,), jnp.float32),
            pltpu.SemaphoreType.DMA,
            pltpu.SemaphoreType.DMA,
        ],
    )(xt, meta_p, bounds_p)


def kernel(X, S_rows, S_cols, S_vals):
    batch, n_items = X.shape
    nnz = S_rows.shape[0]
    nblk = n_items // _R

    xt = X.T

    npad = -nnz % _M
    cols_p = jnp.pad(S_cols, (0, npad))
    rows_p = jnp.pad(S_rows, (0, npad), constant_values=n_items)
    vals_p = jnp.pad(S_vals, (0, npad))
    meta_p = jnp.stack([cols_p.astype(jnp.int32),
                        rows_p.astype(jnp.int32),
                        lax.bitcast_convert_type(vals_p, jnp.int32)])

    grid = (jnp.arange(nblk + 1, dtype=jnp.int32) * _R).astype(S_rows.dtype)
    bounds = jnp.searchsorted(S_rows, grid, side="left").astype(jnp.int32)
    bpad = (-(nblk + 1)) % 16
    bounds_p = jnp.pad(bounds, (0, bpad), constant_values=nnz)

    out_flat = _sc_spmm(xt, meta_p, bounds_p, n_items, batch, nblk)
    return out_flat.reshape(n_items, batch).T


# speedup vs baseline: 5.4472x; 1.0664x over previous
import functools

import jax
import jax.numpy as jnp
from jax import lax
from jax.experimental import pallas as pl
from jax.experimental.pallas import tpu as pltpu
from jax.experimental.pallas import tpu_sc as plsc

_NC = 2
_NS = 16
_NW = _NC * _NS
_L = 16

_R = 128
_K = 64
_M = 1024
_NK = _M // _K


def _sc_body(bpw, jch, batch,
             xt_hbm, meta_hbm, bounds_hbm, out_hbm,
             bounds_v, meta_v, xr0, xr1, rbase_v, acc_v, g0, g1):
    cid = lax.axis_index("c")
    sid = lax.axis_index("s")
    wid = sid * _NC + cid

    pltpu.sync_copy(bounds_hbm, bounds_v)

    iota16 = lax.iota(jnp.int32, _L)
    zeros16 = jnp.zeros((_L,), jnp.float32)
    xr = (xr0, xr1)
    gsem = (g0, g1)

    def issue_gather(k, buf):
        pltpu.async_copy(
            xt_hbm.at[meta_v.at[0, pl.ds(k * _K, _K)]], xr[buf], gsem[buf])

    def wait_gather(k, buf):
        pltpu.make_async_copy(
            xt_hbm.at[meta_v.at[0, pl.ds(k * _K, _K)]], xr[buf],
            gsem[buf]).wait()

    def block_body(bi, carry):
        blk = wid * bpw + bi
        base_row = blk * _R

        def zbody(i, zc):
            acc_v[pl.ds(i * _L, _L)] = zeros16
            return zc
        lax.fori_loop(0, _R * batch // _L, zbody, 0)

        eb0 = plsc.load_gather(bounds_v, [jnp.full((_L,), blk, jnp.int32)])
        eb1 = plsc.load_gather(bounds_v, [jnp.full((_L,), blk + 1, jnp.int32)])
        e_start = jnp.max(eb0)
        e_end = jnp.max(eb1)
        m0 = e_start // _M
        m1 = (e_end + _M - 1) // _M

        def group_body(m, gc):
            mbase = m * _M
            pltpu.sync_copy(meta_hbm.at[:, pl.ds(mbase, _M)], meta_v)

            def live(k):
                return jnp.logical_and(mbase + k * _K < e_end,
                                       mbase + (k + 1) * _K > e_start)

            pl.when(live(0))(lambda: issue_gather(0, 0))

            for k in range(_NK):
                buf = k & 1
                if k + 1 < _NK:
                    pl.when(live(k + 1))(
                        functools.partial(issue_gather, k + 1, 1 - buf))

                def do_chunk(k=k, buf=buf):
                    wait_gather(k, buf)
                    xrb = xr[buf]

                    for g in range(_K // _L):
                        rows16 = meta_v[1, pl.ds(k * _K + g * _L, _L)]
                        ge16 = (mbase + k * _K + g * _L) + iota16
                        ok = jnp.logical_and(ge16 >= e_start, ge16 < e_end)
                        rloc = jnp.clip(rows16 - base_row, 0, _R - 1)
                        rbase_v[pl.ds(g * _L, _L)] = (
                            jnp.where(ok, rloc, _R) * batch)

                    half = jch // 2
                    @plsc.parallel_loop(0, 2 * _K, 1, unroll=2)
                    def scale_body(i):
                        e = i >> 1
                        h = (i & 1) * half
                        li = jnp.full((_L,), k * _K + e, jnp.int32)
                        v16i = plsc.load_gather(
                            meta_v, [jnp.full((_L,), 2, jnp.int32), li])
                        v16 = plsc.bitcast(v16i, jnp.float32)
                        b16 = plsc.load_gather(
                            rbase_v, [jnp.full((_L,), e, jnp.int32)])
                        addr0 = b16 + iota16 + (h * _L)
                        ys = [xrb[e, pl.ds((h + j) * _L, _L)] * v16
                              for j in range(half)]
                        for j in range(half):
                            plsc.addupdate_scatter(
                                acc_v, [addr0 + (j * _L)], ys[j])

                pl.when(live(k))(do_chunk)
            return gc
        lax.fori_loop(m0, m1, group_body, 0)

        pltpu.sync_copy(acc_v.at[pl.ds(0, _R * batch)],
                        out_hbm.at[pl.ds(base_row * batch, _R * batch)])
        return carry
    lax.fori_loop(0, bpw, block_body, 0)


@functools.partial(jax.jit, static_argnums=(3, 4, 5))
def _sc_spmm(xt, meta_p, bounds_p, n_items, batch, nblk):
    bpw = nblk // _NW
    jch = batch // _L
    mesh = plsc.VectorSubcoreMesh(core_axis_name="c", subcore_axis_name="s")
    body = functools.partial(_sc_body, bpw, jch, batch)
    return pl.kernel(
        body,
        out_type=jax.ShapeDtypeStruct((n_items * batch,), jnp.float32),
        mesh=mesh,
        compiler_params=pltpu.CompilerParams(needs_layout_passes=False),
        scratch_types=[
            pltpu.VMEM((bounds_p.shape[0],), jnp.int32),
            pltpu.VMEM((3, _M), jnp.int32),
            pltpu.VMEM((_K, batch), jnp.float32),
            pltpu.VMEM((_K, batch), jnp.float32),
            pltpu.VMEM((_K,), jnp.int32),
            pltpu.VMEM(((_R + 1) * batch,), jnp.float32),
            pltpu.SemaphoreType.DMA,
            pltpu.SemaphoreType.DMA,
        ],
    )(xt, meta_p, bounds_p)


def kernel(X, S_rows, S_cols, S_vals):
    batch, n_items = X.shape
    nnz = S_rows.shape[0]
    nblk = n_items // _R

    xt = X.T

    npad = -nnz % _M
    cols_p = jnp.pad(S_cols, (0, npad))
    rows_p = jnp.pad(S_rows, (0, npad), constant_values=n_items)
    vals_p = jnp.pad(S_vals, (0, npad))
    meta_p = jnp.stack([cols_p.astype(jnp.int32),
                        rows_p.astype(jnp.int32),
                        lax.bitcast_convert_type(vals_p, jnp.int32)])

    grid = (jnp.arange(nblk + 1, dtype=jnp.int32) * _R).astype(S_rows.dtype)
    bounds = jnp.searchsorted(S_rows, grid, side="left").astype(jnp.int32)
    bpad = (-(nblk + 1)) % 16
    bounds_p = jnp.pad(bounds, (0, bpad), constant_values=nnz)

    out_flat = _sc_spmm(xt, meta_p, bounds_p, n_items, batch, nblk)
    return out_flat.reshape(n_items, batch).T
